# baseline (speedup 1.0000x reference)
import functools

import jax
import jax.numpy as jnp
import numpy as np
from jax.experimental import pallas as pl
from jax.experimental.pallas import tpu as pltpu

BS = 2
NA = 128
HW = 4096
C = 256
D_FF = 2048
H = 64
SCALE = 1.0 / np.sqrt(C // 8)

J_BLK = 512
H_BLK = 2048

_dot = functools.partial(jnp.dot, preferred_element_type=jnp.float32)


def _proj_scores_kernel(tok_ref, supp_ref, query_ref,
                        wqa_ref, bqa_ref, wks_ref, bks_ref,
                        wka_ref, bka_ref, wvs_ref, bvs_ref,
                        sa_ref, sqc_ref, vsc_ref, vsum_ref):
    b = pl.program_id(0)
    h = pl.program_id(1)
    tok = tok_ref[0]
    supp = supp_ref[0]
    query = query_ref[0]
    qa = _dot(tok, wqa_ref[...]) + bqa_ref[...]
    ka = _dot(tok, wka_ref[...]) + bka_ref[...]
    ks = _dot(supp, wks_ref[...]) + bks_ref[...]
    vs = _dot(supp, wvs_ref[...]) + bvs_ref[...]
    qq = _dot(query, wqa_ref[...]) + bqa_ref[...]
    sa = jax.lax.dot_general(qa, ks, (((1,), (1,)), ((), ())),
                             preferred_element_type=jnp.float32) * SCALE
    sq = jax.lax.dot_general(qq, ka, (((1,), (1,)), ((), ())),
                             preferred_element_type=jnp.float32) * SCALE
    sa_ref[...] = sa
    sqc_ref[...] = jnp.where(b == 1, -sq, sq)
    vsc_ref[...] = vs
    part = jnp.broadcast_to(jnp.sum(vs, axis=0, keepdims=True), (8, C))

    @pl.when(h == 0)
    def _():
        vsum_ref[...] = part

    @pl.when(h != 0)
    def _():
        vsum_ref[...] = vsum_ref[...] + part


def _decode_ffn_kernel(sqc_ref, sa_ref, vsc_ref, vsum_ref,
                       w1_ref, b1_ref, w2_ref, b2_ref,
                       out_ref):
    sqc = sqc_ref[...]
    q0 = jnp.argmax(sqc[:, :NA], axis=1, keepdims=True)
    q1 = jnp.argmin(sqc[:, NA:], axis=1, keepdims=True)
    sac = sa_ref[...]
    a0 = jnp.argmax(sac[:NA, :], axis=0, keepdims=True)
    a1 = jnp.argmax(sac[NA:, :], axis=0, keepdims=True)
    d = _dot(sqc, sac)
    md = d + jnp.where(q0 == a0, 0.0, -1e6) + jnp.where(q1 == a1, 0.0, 1e6)
    t0 = jax.nn.sigmoid(md)
    ab = _dot(t0, vsc_ref[...])
    acc0 = ab[:, :C]
    acc1 = vsum_ref[8:9, :] - ab[:, C:]
    h0 = jnp.maximum(_dot(acc0, w1_ref[...]) + b1_ref[...], 0.0)
    o0 = _dot(h0, w2_ref[...]) + b2_ref[...]
    h1 = jnp.maximum(_dot(acc1, w1_ref[...]) + b1_ref[...], 0.0)
    o1 = _dot(h1, w2_ref[...]) + b2_ref[...]
    out_ref[0] = o0.reshape(J_BLK // 16, 16, C)
    out_ref[1] = o1.reshape(J_BLK // 16, 16, C)


def _flat_shift(yt, s):
    n = yt.shape[0]
    zrow = jnp.zeros((n, 1, 256), jnp.float32)
    if s > 0:
        up = jnp.concatenate([yt[:, 1:, :], zrow], axis=1)
        return jnp.concatenate([yt[:, :, s:], up[:, :, :s]], axis=2)
    elif s < 0:
        dn = jnp.concatenate([zrow, yt[:, :-1, :]], axis=1)
        return jnp.concatenate([dn[:, :, s:], yt[:, :, :s]], axis=2)
    return yt


def _shift_taps(y, stride, col):
    acc = jnp.zeros((stride, 16, 256), jnp.float32)
    for t in range(9):
        ky, kx = t // 3 - 1, t % 3 - 1
        sh = _flat_shift(y[t * stride:(t + 1) * stride], ky * H + kx)
        if kx == 1:
            sh = jnp.where(col == H - 1, 0.0, sh)
        elif kx == -1:
            sh = jnp.where(col == 0, 0.0, sh)
        acc = acc + sh
    return acc


def _conv_kernel(x_ref, w3_ref, w1_ref, out_ref):
    col = jax.lax.broadcasted_iota(jnp.int32, (1, 1, 256), 2) % H
    y3 = jnp.concatenate(
        [_dot(w3_ref[...], x_ref[0, :, r, :]).reshape(9 * (C // 8), 1, C)
         for r in range(16)], axis=1)
    z = jnp.maximum(_shift_taps(y3, C // 8, col), 0.0)
    y1 = jnp.concatenate(
        [_dot(w1_ref[...], z[:, r, :]).reshape(9 * 8, 1, C)
         for r in range(16)], axis=1)
    out_ref[0] = _shift_taps(y1, 8, col)[:3]


def kernel(tok_agent, enc_feat_supp, enc_feat_query,
           Wqa, bqa, Wks, bks, Wka, bka, Wvs, bvs,
           W1, b1, W2, b2, conv3_w, conv1_w, *, interpret=False):
    n_h = HW // H_BLK
    b2d = lambda v: v.reshape(1, -1)
    sa, sqc, vsc, vsum = pl.pallas_call(
        _proj_scores_kernel,
        grid=(BS, n_h),
        in_specs=[
            pl.BlockSpec((1, NA, C), lambda b, h: (b, 0, 0)),
            pl.BlockSpec((1, H_BLK, C), lambda b, h: (b, h, 0)),
            pl.BlockSpec((1, H_BLK, C), lambda b, h: (b, h, 0)),
            pl.BlockSpec((C, C), lambda b, h: (0, 0)),
            pl.BlockSpec((1, C), lambda b, h: (0, 0)),
            pl.BlockSpec((C, C), lambda b, h: (0, 0)),
            pl.BlockSpec((1, C), lambda b, h: (0, 0)),
            pl.BlockSpec((C, C), lambda b, h: (0, 0)),
            pl.BlockSpec((1, C), lambda b, h: (0, 0)),
            pl.BlockSpec((C, C), lambda b, h: (0, 0)),
            pl.BlockSpec((1, C), lambda b, h: (0, 0)),
        ],
        out_specs=[
            pl.BlockSpec((NA, H_BLK), lambda b, h: (b, h)),
            pl.BlockSpec((H_BLK, NA), lambda b, h: (h, b)),
            pl.BlockSpec((H_BLK, C), lambda b, h: (h, b)),
            pl.BlockSpec((8, C), lambda b, h: (b, 0)),
        ],
        out_shape=[
            jax.ShapeDtypeStruct((BS * NA, HW), jnp.float32),
            jax.ShapeDtypeStruct((HW, BS * NA), jnp.float32),
            jax.ShapeDtypeStruct((HW, BS * C), jnp.float32),
            jax.ShapeDtypeStruct((BS * 8, C), jnp.float32),
        ],
        compiler_params=pltpu.CompilerParams(
            dimension_semantics=("parallel", "arbitrary"),
            vmem_limit_bytes=56 * 1024 * 1024,
        ),
        name="proj_scores",
        interpret=interpret,
    )(tok_agent, enc_feat_supp, enc_feat_query,
      Wqa, b2d(bqa), Wks, b2d(bks), Wka, b2d(bka), Wvs, b2d(bvs))

    n_j = HW // J_BLK
    ffn_out = pl.pallas_call(
        _decode_ffn_kernel,
        grid=(n_j,),
        in_specs=[
            pl.BlockSpec((J_BLK, BS * NA), lambda j: (j, 0)),
            pl.BlockSpec((BS * NA, HW), lambda j: (0, 0)),
            pl.BlockSpec((HW, BS * C), lambda j: (0, 0)),
            pl.BlockSpec((BS * 8, C), lambda j: (0, 0)),
            pl.BlockSpec((C, D_FF), lambda j: (0, 0)),
            pl.BlockSpec((1, D_FF), lambda j: (0, 0)),
            pl.BlockSpec((D_FF, C), lambda j: (0, 0)),
            pl.BlockSpec((1, C), lambda j: (0, 0)),
        ],
        out_specs=pl.BlockSpec((BS, J_BLK // 16, 16, C), lambda j: (0, j, 0, 0)),
        out_shape=jax.ShapeDtypeStruct((BS, C, 16, C), jnp.float32),
        compiler_params=pltpu.CompilerParams(
            dimension_semantics=("parallel",),
            vmem_limit_bytes=56 * 1024 * 1024,
        ),
        name="decode_ffn",
        interpret=interpret,
    )(sqc, sa, vsc, vsum, W1, b2d(b1), W2, b2d(b2))

    w3 = conv3_w.transpose(2, 3, 0, 1).reshape(9 * (C // 8), C)
    w1c = jnp.pad(conv1_w.transpose(2, 3, 0, 1).reshape(9, 3, C // 8),
                  ((0, 0), (0, 5), (0, 0))).reshape(9 * 8, C // 8)
    out = pl.pallas_call(
        _conv_kernel,
        grid=(BS,),
        in_specs=[
            pl.BlockSpec((1, C, 16, C), lambda b: (b, 0, 0, 0)),
            pl.BlockSpec((9 * (C // 8), C), lambda b: (0, 0)),
            pl.BlockSpec((9 * 8, C // 8), lambda b: (0, 0)),
        ],
        out_specs=pl.BlockSpec((1, 3, 16, C), lambda b: (b, 0, 0, 0)),
        out_shape=jax.ShapeDtypeStruct((BS, 3, 16, C), jnp.float32),
        compiler_params=pltpu.CompilerParams(
            dimension_semantics=("parallel",),
            vmem_limit_bytes=56 * 1024 * 1024,
        ),
        name="conv_head",
        interpret=interpret,
    )(ffn_out, w3, w1c)
    return out.reshape(BS, 3, H, H)


# speedup vs baseline: 1.0864x; 1.0864x over previous
import functools

import jax
import jax.numpy as jnp
import numpy as np
from jax.experimental import pallas as pl
from jax.experimental.pallas import tpu as pltpu

BS = 2
NA = 128
HW = 4096
C = 256
D_FF = 2048
H = 64
SCALE = 1.0 / np.sqrt(C // 8)

J_BLK = 512
H_BLK = 2048

_dot = functools.partial(jnp.dot, preferred_element_type=jnp.float32)


def _proj_scores_kernel(tok_ref, supp_ref, query_ref,
                        wqa_ref, bqa_ref, wks_ref, bks_ref,
                        wka_ref, bka_ref, wvs_ref, bvs_ref,
                        sa_ref, sqc_ref, vsc_ref, vsum_ref):
    b = pl.program_id(0)
    h = pl.program_id(1)
    tok = tok_ref[0]
    supp = supp_ref[0]
    query = query_ref[0]
    qa = _dot(tok, wqa_ref[...]) + bqa_ref[...]
    ka = _dot(tok, wka_ref[...]) + bka_ref[...]
    ks = _dot(supp, wks_ref[...]) + bks_ref[...]
    vs = _dot(supp, wvs_ref[...]) + bvs_ref[...]
    qq = _dot(query, wqa_ref[...]) + bqa_ref[...]
    sa = jax.lax.dot_general(qa, ks, (((1,), (1,)), ((), ())),
                             preferred_element_type=jnp.float32) * SCALE
    sq = jax.lax.dot_general(qq, ka, (((1,), (1,)), ((), ())),
                             preferred_element_type=jnp.float32) * SCALE
    sa_ref[...] = sa
    sqc_ref[...] = jnp.where(b == 1, -sq, sq)
    vsc_ref[...] = vs
    part = jnp.broadcast_to(jnp.sum(vs, axis=0, keepdims=True), (8, C))

    @pl.when(h == 0)
    def _():
        vsum_ref[...] = part

    @pl.when(h != 0)
    def _():
        vsum_ref[...] = vsum_ref[...] + part


def _decode_ffn_kernel(sqc_ref, sa_ref, vsc_ref, vsum_ref,
                       w1_ref, b1_ref, w2_ref, b2_ref,
                       out_ref):
    sqc = sqc_ref[...]
    q0 = jnp.argmax(sqc[:, :NA], axis=1, keepdims=True)
    q1 = jnp.argmin(sqc[:, NA:], axis=1, keepdims=True)
    sac = sa_ref[...]
    a0 = jnp.argmax(sac[:NA, :], axis=0, keepdims=True)
    a1 = jnp.argmax(sac[NA:, :], axis=0, keepdims=True)
    d = _dot(sqc, sac)
    md = d + jnp.where(q0 == a0, 0.0, -1e6) + jnp.where(q1 == a1, 0.0, 1e6)
    t0 = jax.nn.sigmoid(md)
    ab = _dot(t0, vsc_ref[...])
    acc0 = ab[:, :C]
    acc1 = vsum_ref[8:9, :] - ab[:, C:]
    h0 = jnp.maximum(_dot(acc0, w1_ref[...]) + b1_ref[...], 0.0)
    o0 = (_dot(h0, w2_ref[...]) + b2_ref[...]).reshape(J_BLK // 16, 16, C)
    h1 = jnp.maximum(_dot(acc1, w1_ref[...]) + b1_ref[...], 0.0)
    o1 = (_dot(h1, w2_ref[...]) + b2_ref[...]).reshape(J_BLK // 16, 16, C)
    for r in range(16):
        out_ref[0, :, r * C:(r + 1) * C] = o0[:, r, :]
        out_ref[1, :, r * C:(r + 1) * C] = o1[:, r, :]


def _shift_taps(y, stride, col):
    acc = jnp.zeros((stride, HW), jnp.float32)
    for t in range(9):
        ky, kx = t // 3 - 1, t % 3 - 1
        s = ky * H + kx
        yt = y[t * stride:(t + 1) * stride, :]
        if s > 0:
            sh = jnp.concatenate(
                [yt[:, s:], jnp.zeros((stride, s), jnp.float32)], axis=1)
        elif s < 0:
            sh = jnp.concatenate(
                [jnp.zeros((stride, -s), jnp.float32), yt[:, :HW + s]], axis=1)
        else:
            sh = yt
        if kx == 1:
            sh = jnp.where(col == H - 1, 0.0, sh)
        elif kx == -1:
            sh = jnp.where(col == 0, 0.0, sh)
        acc = acc + sh
    return acc


def _conv_kernel(x_ref, w3_ref, w1_ref, out_ref):
    x = x_ref[0]
    col = jax.lax.broadcasted_iota(jnp.int32, (1, HW), 1) % H
    y3 = _dot(w3_ref[...], x)
    z = jnp.maximum(_shift_taps(y3, C // 8, col), 0.0)
    y1 = _dot(w1_ref[...], z)
    out_ref[0] = _shift_taps(y1, 8, col)[:3, :]


def kernel(tok_agent, enc_feat_supp, enc_feat_query,
           Wqa, bqa, Wks, bks, Wka, bka, Wvs, bvs,
           W1, b1, W2, b2, conv3_w, conv1_w, *, interpret=False):
    n_h = HW // H_BLK
    b2d = lambda v: v.reshape(1, -1)
    sa, sqc, vsc, vsum = pl.pallas_call(
        _proj_scores_kernel,
        grid=(BS, n_h),
        in_specs=[
            pl.BlockSpec((1, NA, C), lambda b, h: (b, 0, 0)),
            pl.BlockSpec((1, H_BLK, C), lambda b, h: (b, h, 0)),
            pl.BlockSpec((1, H_BLK, C), lambda b, h: (b, h, 0)),
            pl.BlockSpec((C, C), lambda b, h: (0, 0)),
            pl.BlockSpec((1, C), lambda b, h: (0, 0)),
            pl.BlockSpec((C, C), lambda b, h: (0, 0)),
            pl.BlockSpec((1, C), lambda b, h: (0, 0)),
            pl.BlockSpec((C, C), lambda b, h: (0, 0)),
            pl.BlockSpec((1, C), lambda b, h: (0, 0)),
            pl.BlockSpec((C, C), lambda b, h: (0, 0)),
            pl.BlockSpec((1, C), lambda b, h: (0, 0)),
        ],
        out_specs=[
            pl.BlockSpec((NA, H_BLK), lambda b, h: (b, h)),
            pl.BlockSpec((H_BLK, NA), lambda b, h: (h, b)),
            pl.BlockSpec((H_BLK, C), lambda b, h: (h, b)),
            pl.BlockSpec((8, C), lambda b, h: (b, 0)),
        ],
        out_shape=[
            jax.ShapeDtypeStruct((BS * NA, HW), jnp.float32),
            jax.ShapeDtypeStruct((HW, BS * NA), jnp.float32),
            jax.ShapeDtypeStruct((HW, BS * C), jnp.float32),
            jax.ShapeDtypeStruct((BS * 8, C), jnp.float32),
        ],
        compiler_params=pltpu.CompilerParams(
            dimension_semantics=("parallel", "arbitrary"),
            vmem_limit_bytes=56 * 1024 * 1024,
        ),
        name="proj_scores",
        interpret=interpret,
    )(tok_agent, enc_feat_supp, enc_feat_query,
      Wqa, b2d(bqa), Wks, b2d(bks), Wka, b2d(bka), Wvs, b2d(bvs))

    n_j = HW // J_BLK
    ffn_out = pl.pallas_call(
        _decode_ffn_kernel,
        grid=(n_j,),
        in_specs=[
            pl.BlockSpec((J_BLK, BS * NA), lambda j: (j, 0)),
            pl.BlockSpec((BS * NA, HW), lambda j: (0, 0)),
            pl.BlockSpec((HW, BS * C), lambda j: (0, 0)),
            pl.BlockSpec((BS * 8, C), lambda j: (0, 0)),
            pl.BlockSpec((C, D_FF), lambda j: (0, 0)),
            pl.BlockSpec((1, D_FF), lambda j: (0, 0)),
            pl.BlockSpec((D_FF, C), lambda j: (0, 0)),
            pl.BlockSpec((1, C), lambda j: (0, 0)),
        ],
        out_specs=pl.BlockSpec((BS, J_BLK // 16, HW), lambda j: (0, j, 0)),
        out_shape=jax.ShapeDtypeStruct((BS, C, HW), jnp.float32),
        compiler_params=pltpu.CompilerParams(
            dimension_semantics=("parallel",),
            vmem_limit_bytes=56 * 1024 * 1024,
        ),
        name="decode_ffn",
        interpret=interpret,
    )(sqc, sa, vsc, vsum, W1, b2d(b1), W2, b2d(b2))

    w3 = conv3_w.transpose(2, 3, 0, 1).reshape(9 * (C // 8), C)
    w1c = jnp.pad(conv1_w.transpose(2, 3, 0, 1).reshape(9, 3, C // 8),
                  ((0, 0), (0, 5), (0, 0))).reshape(9 * 8, C // 8)
    out = pl.pallas_call(
        _conv_kernel,
        grid=(BS,),
        in_specs=[
            pl.BlockSpec((1, C, HW), lambda b: (b, 0, 0)),
            pl.BlockSpec((9 * (C // 8), C), lambda b: (0, 0)),
            pl.BlockSpec((9 * 8, C // 8), lambda b: (0, 0)),
        ],
        out_specs=pl.BlockSpec((1, 3, HW), lambda b: (b, 0, 0)),
        out_shape=jax.ShapeDtypeStruct((BS, 3, HW), jnp.float32),
        compiler_params=pltpu.CompilerParams(
            dimension_semantics=("parallel",),
            vmem_limit_bytes=56 * 1024 * 1024,
        ),
        name="conv_head",
        interpret=interpret,
    )(ffn_out, w3, w1c)
    return out.reshape(BS, 3, H, H)


# speedup vs baseline: 1.1817x; 1.0877x over previous
import functools

import jax
import jax.numpy as jnp
import numpy as np
from jax.experimental import pallas as pl
from jax.experimental.pallas import tpu as pltpu

BS = 2
NA = 128
HW = 4096
C = 256
D_FF = 2048
H = 64
SCALE = 1.0 / np.sqrt(C // 8)

J_BLK = 512

_dot = functools.partial(jnp.dot, preferred_element_type=jnp.float32)


def _decode_ffn_kernel(tok_ref, supp_ref, query_ref,
                       wqa_ref, bqa_ref, wks_ref, bks_ref,
                       wka_ref, bka_ref, wvs_ref, bvs_ref,
                       w1_ref, b1_ref, w2_ref, b2_ref,
                       out_ref, sa_s, vsc_s, vsum_s):
    j = pl.program_id(0)

    @pl.when(j == 0)
    def _():
        for b in range(BS):
            qa = _dot(tok_ref[b], wqa_ref[...]) + bqa_ref[...]
            ks = _dot(supp_ref[b], wks_ref[...]) + bks_ref[...]
            vs = _dot(supp_ref[b], wvs_ref[...]) + bvs_ref[...]
            sa_s[b * NA:(b + 1) * NA, :] = jax.lax.dot_general(
                qa, ks, (((1,), (1,)), ((), ())),
                preferred_element_type=jnp.float32) * SCALE
            vsc_s[:, b * C:(b + 1) * C] = vs
            if b == 1:
                vsum_s[...] = jnp.broadcast_to(
                    jnp.sum(vs, axis=0, keepdims=True), (8, C))

    ka0 = _dot(tok_ref[0], wka_ref[...]) + bka_ref[...]
    ka1 = _dot(tok_ref[1], wka_ref[...]) + bka_ref[...]
    qq0 = _dot(query_ref[0], wqa_ref[...]) + bqa_ref[...]
    qq1 = _dot(query_ref[1], wqa_ref[...]) + bqa_ref[...]
    sq0 = jax.lax.dot_general(qq0, ka0, (((1,), (1,)), ((), ())),
                              preferred_element_type=jnp.float32) * SCALE
    sq1 = jax.lax.dot_general(qq1, ka1, (((1,), (1,)), ((), ())),
                              preferred_element_type=jnp.float32) * SCALE
    sqc = jnp.concatenate([sq0, -sq1], axis=1)
    q0 = jnp.argmax(sq0, axis=1, keepdims=True)
    q1 = jnp.argmax(sq1, axis=1, keepdims=True)
    sac = sa_s[...]
    a0 = jnp.argmax(sac[:NA, :], axis=0, keepdims=True)
    a1 = jnp.argmax(sac[NA:, :], axis=0, keepdims=True)
    d = _dot(sqc, sac)
    md = d + jnp.where(q0 == a0, 0.0, -1e6) + jnp.where(q1 == a1, 0.0, 1e6)
    t0 = jax.nn.sigmoid(md)
    ab = _dot(t0, vsc_s[...])
    acc0 = ab[:, :C]
    acc1 = vsum_s[0:1, :] - ab[:, C:]
    h0 = jnp.maximum(_dot(acc0, w1_ref[...]) + b1_ref[...], 0.0)
    o0 = (_dot(h0, w2_ref[...]) + b2_ref[...]).reshape(J_BLK // 16, 16, C)
    h1 = jnp.maximum(_dot(acc1, w1_ref[...]) + b1_ref[...], 0.0)
    o1 = (_dot(h1, w2_ref[...]) + b2_ref[...]).reshape(J_BLK // 16, 16, C)
    for r in range(16):
        out_ref[0, :, r * C:(r + 1) * C] = o0[:, r, :]
        out_ref[1, :, r * C:(r + 1) * C] = o1[:, r, :]


def _shift_taps(y, stride, col):
    acc = jnp.zeros((stride, HW), jnp.float32)
    for t in range(9):
        ky, kx = t // 3 - 1, t % 3 - 1
        s = ky * H + kx
        yt = y[t * stride:(t + 1) * stride, :]
        if s > 0:
            sh = jnp.concatenate(
                [yt[:, s:], jnp.zeros((stride, s), jnp.float32)], axis=1)
        elif s < 0:
            sh = jnp.concatenate(
                [jnp.zeros((stride, -s), jnp.float32), yt[:, :HW + s]], axis=1)
        else:
            sh = yt
        if kx == 1:
            sh = jnp.where(col == H - 1, 0.0, sh)
        elif kx == -1:
            sh = jnp.where(col == 0, 0.0, sh)
        acc = acc + sh
    return acc


def _conv_kernel(x_ref, w3_ref, w1_ref, out_ref):
    x = x_ref[0]
    col = jax.lax.broadcasted_iota(jnp.int32, (1, HW), 1) % H
    y3 = _dot(w3_ref[...], x)
    z = jnp.maximum(_shift_taps(y3, C // 8, col), 0.0)
    y1 = _dot(w1_ref[...], z)
    out_ref[0] = _shift_taps(y1, 8, col)[:3, :]


def kernel(tok_agent, enc_feat_supp, enc_feat_query,
           Wqa, bqa, Wks, bks, Wka, bka, Wvs, bvs,
           W1, b1, W2, b2, conv3_w, conv1_w, *, interpret=False):
    b2d = lambda v: v.reshape(1, -1)
    n_j = HW // J_BLK
    wspec = pl.BlockSpec((C, C), lambda j: (0, 0))
    bspec = pl.BlockSpec((1, C), lambda j: (0, 0))
    ffn_out = pl.pallas_call(
        _decode_ffn_kernel,
        grid=(n_j,),
        in_specs=[
            pl.BlockSpec((BS, NA, C), lambda j: (0, 0, 0)),
            pl.BlockSpec((BS, HW, C), lambda j: (0, 0, 0)),
            pl.BlockSpec((BS, J_BLK, C), lambda j: (0, j, 0)),
            wspec, bspec, wspec, bspec, wspec, bspec, wspec, bspec,
            pl.BlockSpec((C, D_FF), lambda j: (0, 0)),
            pl.BlockSpec((1, D_FF), lambda j: (0, 0)),
            pl.BlockSpec((D_FF, C), lambda j: (0, 0)),
            pl.BlockSpec((1, C), lambda j: (0, 0)),
        ],
        out_specs=pl.BlockSpec((BS, J_BLK // 16, HW), lambda j: (0, j, 0)),
        out_shape=jax.ShapeDtypeStruct((BS, C, HW), jnp.float32),
        scratch_shapes=[
            pltpu.VMEM((BS * NA, HW), jnp.float32),
            pltpu.VMEM((HW, BS * C), jnp.float32),
            pltpu.VMEM((8, C), jnp.float32),
        ],
        compiler_params=pltpu.CompilerParams(
            dimension_semantics=("arbitrary",),
            vmem_limit_bytes=56 * 1024 * 1024,
        ),
        name="decode_ffn",
        interpret=interpret,
    )(tok_agent, enc_feat_supp, enc_feat_query,
      Wqa, b2d(bqa), Wks, b2d(bks), Wka, b2d(bka), Wvs, b2d(bvs),
      W1, b2d(b1), W2, b2d(b2))

    w3 = conv3_w.transpose(2, 3, 0, 1).reshape(9 * (C // 8), C)
    w1c = jnp.pad(conv1_w.transpose(2, 3, 0, 1).reshape(9, 3, C // 8),
                  ((0, 0), (0, 5), (0, 0))).reshape(9 * 8, C // 8)
    out = pl.pallas_call(
        _conv_kernel,
        grid=(BS,),
        in_specs=[
            pl.BlockSpec((1, C, HW), lambda b: (b, 0, 0)),
            pl.BlockSpec((9 * (C // 8), C), lambda b: (0, 0)),
            pl.BlockSpec((9 * 8, C // 8), lambda b: (0, 0)),
        ],
        out_specs=pl.BlockSpec((1, 3, HW), lambda b: (b, 0, 0)),
        out_shape=jax.ShapeDtypeStruct((BS, 3, HW), jnp.float32),
        compiler_params=pltpu.CompilerParams(
            dimension_semantics=("parallel",),
            vmem_limit_bytes=56 * 1024 * 1024,
        ),
        name="conv_head",
        interpret=interpret,
    )(ffn_out, w3, w1c)
    return out.reshape(BS, 3, H, H)


# speedup vs baseline: 1.2708x; 1.0753x over previous
import functools

import jax
import jax.numpy as jnp
import numpy as np
from jax.experimental import pallas as pl
from jax.experimental.pallas import tpu as pltpu

BS = 2
NA = 128
HW = 4096
C = 256
D_FF = 2048
H = 64
SCALE = 1.0 / np.sqrt(C // 8)

J_BLK = 512

_dot = functools.partial(jnp.dot, preferred_element_type=jnp.float32)


def _decode_ffn_kernel(tok_ref, supp_ref, query_ref,
                       wqa_ref, bqa_ref, wks_ref, bks_ref,
                       wka_ref, bka_ref, wvs_ref, bvs_ref,
                       w1_ref, b1_ref, w2_ref, b2_ref,
                       out_ref, sa_s, vsc_s, vsum_s, am_s, ka_s):
    j = pl.program_id(0)

    @pl.when(j == 0)
    def _():
        for b in range(BS):
            qa = _dot(tok_ref[b], wqa_ref[...]) + bqa_ref[...]
            ks = _dot(supp_ref[b], wks_ref[...]) + bks_ref[...]
            vs = _dot(supp_ref[b], wvs_ref[...]) + bvs_ref[...]
            sa_s[b * NA:(b + 1) * NA, :] = jax.lax.dot_general(
                qa, ks, (((1,), (1,)), ((), ())),
                preferred_element_type=jnp.float32) * SCALE
            vsc_s[:, b * C:(b + 1) * C] = vs
            ka_s[b * NA:(b + 1) * NA, :] = (
                _dot(tok_ref[b], wka_ref[...]) + bka_ref[...])
            if b == 1:
                vsum_s[...] = jnp.broadcast_to(
                    jnp.sum(vs, axis=0, keepdims=True), (8, C))
        sac0 = sa_s[...]
        am_s[0:1, :] = jnp.argmax(sac0[:NA, :], axis=0, keepdims=True)
        am_s[1:2, :] = jnp.argmax(sac0[NA:, :], axis=0, keepdims=True)

    ka0 = ka_s[:NA, :]
    ka1 = ka_s[NA:, :]
    qq0 = _dot(query_ref[0], wqa_ref[...]) + bqa_ref[...]
    qq1 = _dot(query_ref[1], wqa_ref[...]) + bqa_ref[...]
    sq0 = jax.lax.dot_general(qq0, ka0, (((1,), (1,)), ((), ())),
                              preferred_element_type=jnp.float32) * SCALE
    sq1 = jax.lax.dot_general(qq1, ka1, (((1,), (1,)), ((), ())),
                              preferred_element_type=jnp.float32) * SCALE
    sqc = jnp.concatenate([sq0, -sq1], axis=1)
    q0 = jnp.argmax(sq0, axis=1, keepdims=True)
    q1 = jnp.argmax(sq1, axis=1, keepdims=True)
    a0 = am_s[0:1, :]
    a1 = am_s[1:2, :]
    d = _dot(sqc, sa_s[...])
    eq0 = q0 == a0
    eq1 = q1 == a1
    s = jax.nn.sigmoid(d)
    t0 = jnp.where(eq0, jnp.where(eq1, s, 1.0), jnp.where(eq1, 0.0, s))
    ab = _dot(t0, vsc_s[...])
    acc0 = ab[:, :C]
    acc1 = vsum_s[0:1, :] - ab[:, C:]
    h0 = jnp.maximum(_dot(acc0, w1_ref[...]) + b1_ref[...], 0.0)
    o0 = (_dot(h0, w2_ref[...]) + b2_ref[...]).reshape(J_BLK // 16, 16, C)
    h1 = jnp.maximum(_dot(acc1, w1_ref[...]) + b1_ref[...], 0.0)
    o1 = (_dot(h1, w2_ref[...]) + b2_ref[...]).reshape(J_BLK // 16, 16, C)
    for r in range(16):
        out_ref[0, :, r * C:(r + 1) * C] = o0[:, r, :]
        out_ref[1, :, r * C:(r + 1) * C] = o1[:, r, :]


def _shift_taps(y, stride, col):
    acc = jnp.zeros((stride, HW), jnp.float32)
    for t in range(9):
        ky, kx = t // 3 - 1, t % 3 - 1
        s = ky * H + kx
        yt = y[t * stride:(t + 1) * stride, :]
        if s > 0:
            sh = jnp.concatenate(
                [yt[:, s:], jnp.zeros((stride, s), jnp.float32)], axis=1)
        elif s < 0:
            sh = jnp.concatenate(
                [jnp.zeros((stride, -s), jnp.float32), yt[:, :HW + s]], axis=1)
        else:
            sh = yt
        if kx == 1:
            sh = jnp.where(col == H - 1, 0.0, sh)
        elif kx == -1:
            sh = jnp.where(col == 0, 0.0, sh)
        acc = acc + sh
    return acc


def _conv_kernel(x_ref, w3_ref, w1_ref, out_ref):
    x = x_ref[0]
    col = jax.lax.broadcasted_iota(jnp.int32, (1, HW), 1) % H
    y3 = _dot(w3_ref[...], x)
    z = jnp.maximum(_shift_taps(y3, C // 8, col), 0.0)
    y1 = _dot(w1_ref[...], z)
    out_ref[0] = _shift_taps(y1, 8, col)[:3, :]


def kernel(tok_agent, enc_feat_supp, enc_feat_query,
           Wqa, bqa, Wks, bks, Wka, bka, Wvs, bvs,
           W1, b1, W2, b2, conv3_w, conv1_w, *, interpret=False):
    b2d = lambda v: v.reshape(1, -1)
    n_j = HW // J_BLK
    wspec = pl.BlockSpec((C, C), lambda j: (0, 0))
    bspec = pl.BlockSpec((1, C), lambda j: (0, 0))
    ffn_out = pl.pallas_call(
        _decode_ffn_kernel,
        grid=(n_j,),
        in_specs=[
            pl.BlockSpec((BS, NA, C), lambda j: (0, 0, 0)),
            pl.BlockSpec((BS, HW, C), lambda j: (0, 0, 0)),
            pl.BlockSpec((BS, J_BLK, C), lambda j: (0, j, 0)),
            wspec, bspec, wspec, bspec, wspec, bspec, wspec, bspec,
            pl.BlockSpec((C, D_FF), lambda j: (0, 0)),
            pl.BlockSpec((1, D_FF), lambda j: (0, 0)),
            pl.BlockSpec((D_FF, C), lambda j: (0, 0)),
            pl.BlockSpec((1, C), lambda j: (0, 0)),
        ],
        out_specs=pl.BlockSpec((BS, J_BLK // 16, HW), lambda j: (0, j, 0)),
        out_shape=jax.ShapeDtypeStruct((BS, C, HW), jnp.float32),
        scratch_shapes=[
            pltpu.VMEM((BS * NA, HW), jnp.float32),
            pltpu.VMEM((HW, BS * C), jnp.float32),
            pltpu.VMEM((8, C), jnp.float32),
            pltpu.VMEM((8, HW), jnp.int32),
            pltpu.VMEM((BS * NA, C), jnp.float32),
        ],
        compiler_params=pltpu.CompilerParams(
            dimension_semantics=("arbitrary",),
            vmem_limit_bytes=56 * 1024 * 1024,
        ),
        name="decode_ffn",
        interpret=interpret,
    )(tok_agent, enc_feat_supp, enc_feat_query,
      Wqa, b2d(bqa), Wks, b2d(bks), Wka, b2d(bka), Wvs, b2d(bvs),
      W1, b2d(b1), W2, b2d(b2))

    w3 = conv3_w.transpose(2, 3, 0, 1).reshape(9 * (C // 8), C)
    w1c = jnp.pad(conv1_w.transpose(2, 3, 0, 1).reshape(9, 3, C // 8),
                  ((0, 0), (0, 5), (0, 0))).reshape(9 * 8, C // 8)
    out = pl.pallas_call(
        _conv_kernel,
        grid=(BS,),
        in_specs=[
            pl.BlockSpec((1, C, HW), lambda b: (b, 0, 0)),
            pl.BlockSpec((9 * (C // 8), C), lambda b: (0, 0)),
            pl.BlockSpec((9 * 8, C // 8), lambda b: (0, 0)),
        ],
        out_specs=pl.BlockSpec((1, 3, HW), lambda b: (b, 0, 0)),
        out_shape=jax.ShapeDtypeStruct((BS, 3, HW), jnp.float32),
        compiler_params=pltpu.CompilerParams(
            dimension_semantics=("parallel",),
            vmem_limit_bytes=56 * 1024 * 1024,
        ),
        name="conv_head",
        interpret=interpret,
    )(ffn_out, w3, w1c)
    return out.reshape(BS, 3, H, H)


# speedup vs baseline: 1.3187x; 1.0377x over previous
import functools

import jax
import jax.numpy as jnp
import numpy as np
from jax.experimental import pallas as pl
from jax.experimental.pallas import tpu as pltpu

BS = 2
NA = 128
HW = 4096
C = 256
D_FF = 2048
H = 64
SCALE = 1.0 / np.sqrt(C // 8)

J_BLK = 512

_dot = functools.partial(jnp.dot, preferred_element_type=jnp.float32)


def _decode_ffn_kernel(tok_ref, supp_ref, query_ref,
                       wqa_ref, bqa_ref, wks_ref, bks_ref,
                       wka_ref, bka_ref, wvs_ref, bvs_ref,
                       w1_ref, b1_ref, w2_ref, b2_ref, w3_ref, w1c_ref,
                       out_ref, sa_s, vsc_s, vsum_s, am_s, ka_s, dec_s):
    j = pl.program_id(0)

    @pl.when(j == 0)
    def _():
        for b in range(BS):
            qa = _dot(tok_ref[b], wqa_ref[...]) + bqa_ref[...]
            ks = _dot(supp_ref[b], wks_ref[...]) + bks_ref[...]
            vs = _dot(supp_ref[b], wvs_ref[...]) + bvs_ref[...]
            sa_s[b * NA:(b + 1) * NA, :] = jax.lax.dot_general(
                qa, ks, (((1,), (1,)), ((), ())),
                preferred_element_type=jnp.float32) * SCALE
            vsc_s[:, b * C:(b + 1) * C] = vs
            ka_s[b * NA:(b + 1) * NA, :] = (
                _dot(tok_ref[b], wka_ref[...]) + bka_ref[...])
            if b == 1:
                vsum_s[...] = jnp.broadcast_to(
                    jnp.sum(vs, axis=0, keepdims=True), (8, C))
        sac0 = sa_s[...]
        am_s[0:1, :] = jnp.argmax(sac0[:NA, :], axis=0, keepdims=True)
        am_s[1:2, :] = jnp.argmax(sac0[NA:, :], axis=0, keepdims=True)

    ka0 = ka_s[:NA, :]
    ka1 = ka_s[NA:, :]
    qq0 = _dot(query_ref[0], wqa_ref[...]) + bqa_ref[...]
    qq1 = _dot(query_ref[1], wqa_ref[...]) + bqa_ref[...]
    sq0 = jax.lax.dot_general(qq0, ka0, (((1,), (1,)), ((), ())),
                              preferred_element_type=jnp.float32) * SCALE
    sq1 = jax.lax.dot_general(qq1, ka1, (((1,), (1,)), ((), ())),
                              preferred_element_type=jnp.float32) * SCALE
    sqc = jnp.concatenate([sq0, -sq1], axis=1)
    q0 = jnp.argmax(sq0, axis=1, keepdims=True)
    q1 = jnp.argmax(sq1, axis=1, keepdims=True)
    a0 = am_s[0:1, :]
    a1 = am_s[1:2, :]
    d = _dot(sqc, sa_s[...])
    eq0 = q0 == a0
    eq1 = q1 == a1
    s = jax.nn.sigmoid(d)
    t0 = jnp.where(eq0, jnp.where(eq1, s, 1.0), jnp.where(eq1, 0.0, s))
    ab = _dot(t0, vsc_s[...])
    acc0 = ab[:, :C]
    acc1 = vsum_s[0:1, :] - ab[:, C:]
    h0 = jnp.maximum(_dot(acc0, w1_ref[...]) + b1_ref[...], 0.0)
    o0 = (_dot(h0, w2_ref[...]) + b2_ref[...]).reshape(J_BLK // 16, 16, C)
    h1 = jnp.maximum(_dot(acc1, w1_ref[...]) + b1_ref[...], 0.0)
    o1 = (_dot(h1, w2_ref[...]) + b2_ref[...]).reshape(J_BLK // 16, 16, C)
    row = pl.multiple_of(j * (J_BLK // 16), J_BLK // 16)
    for r in range(16):
        dec_s[pl.ds(row, J_BLK // 16), r * C:(r + 1) * C] = o0[:, r, :]
        dec_s[pl.ds(C + row, J_BLK // 16), r * C:(r + 1) * C] = o1[:, r, :]

    @pl.when(j == pl.num_programs(0) - 1)
    def _():
        col = jax.lax.broadcasted_iota(jnp.int32, (1, HW), 1) % H
        for b in range(BS):
            x = dec_s[b * C:(b + 1) * C, :]
            y3 = _dot(w3_ref[...], x)
            z = jnp.maximum(_shift_taps(y3, C // 8, col), 0.0)
            y1 = _dot(w1c_ref[...], z)
            out_ref[b] = _shift_taps(y1, 8, col)[:3, :]


def _shift_taps(y, stride, col):
    acc = jnp.zeros((stride, HW), jnp.float32)
    for t in range(9):
        ky, kx = t // 3 - 1, t % 3 - 1
        s = ky * H + kx
        yt = y[t * stride:(t + 1) * stride, :]
        if s > 0:
            sh = jnp.concatenate(
                [yt[:, s:], jnp.zeros((stride, s), jnp.float32)], axis=1)
        elif s < 0:
            sh = jnp.concatenate(
                [jnp.zeros((stride, -s), jnp.float32), yt[:, :HW + s]], axis=1)
        else:
            sh = yt
        if kx == 1:
            sh = jnp.where(col == H - 1, 0.0, sh)
        elif kx == -1:
            sh = jnp.where(col == 0, 0.0, sh)
        acc = acc + sh
    return acc


def kernel(tok_agent, enc_feat_supp, enc_feat_query,
           Wqa, bqa, Wks, bks, Wka, bka, Wvs, bvs,
           W1, b1, W2, b2, conv3_w, conv1_w, *, interpret=False):
    b2d = lambda v: v.reshape(1, -1)
    n_j = HW // J_BLK
    wspec = pl.BlockSpec((C, C), lambda j: (0, 0))
    bspec = pl.BlockSpec((1, C), lambda j: (0, 0))
    ffn_out = pl.pallas_call(
        _decode_ffn_kernel,
        grid=(n_j,),
        in_specs=[
            pl.BlockSpec((BS, NA, C), lambda j: (0, 0, 0)),
            pl.BlockSpec((BS, HW, C), lambda j: (0, 0, 0)),
            pl.BlockSpec((BS, J_BLK, C), lambda j: (0, j, 0)),
            wspec, bspec, wspec, bspec, wspec, bspec, wspec, bspec,
            pl.BlockSpec((C, D_FF), lambda j: (0, 0)),
            pl.BlockSpec((1, D_FF), lambda j: (0, 0)),
            pl.BlockSpec((D_FF, C), lambda j: (0, 0)),
            pl.BlockSpec((1, C), lambda j: (0, 0)),
            pl.BlockSpec((9 * (C // 8), C), lambda j: (0, 0)),
            pl.BlockSpec((9 * 8, C // 8), lambda j: (0, 0)),
        ],
        out_specs=pl.BlockSpec((BS, 3, HW), lambda j: (0, 0, 0)),
        out_shape=jax.ShapeDtypeStruct((BS, 3, HW), jnp.float32),
        scratch_shapes=[
            pltpu.VMEM((BS * NA, HW), jnp.float32),
            pltpu.VMEM((HW, BS * C), jnp.float32),
            pltpu.VMEM((8, C), jnp.float32),
            pltpu.VMEM((8, HW), jnp.int32),
            pltpu.VMEM((BS * NA, C), jnp.float32),
            pltpu.VMEM((BS * C, HW), jnp.float32),
        ],
        compiler_params=pltpu.CompilerParams(
            dimension_semantics=("arbitrary",),
            vmem_limit_bytes=58 * 1024 * 1024,
        ),
        name="decode_ffn",
        interpret=interpret,
    )(tok_agent, enc_feat_supp, enc_feat_query,
      Wqa, b2d(bqa), Wks, b2d(bks), Wka, b2d(bka), Wvs, b2d(bvs),
      W1, b2d(b1), W2, b2d(b2),
      conv3_w.transpose(2, 3, 0, 1).reshape(9 * (C // 8), C),
      jnp.pad(conv1_w.transpose(2, 3, 0, 1).reshape(9, 3, C // 8),
              ((0, 0), (0, 5), (0, 0))).reshape(9 * 8, C // 8))
    return ffn_out.reshape(BS, 3, H, H)
